# baseline (speedup 1.0000x reference)
import functools

import jax
import jax.numpy as jnp
from jax import lax
from jax.experimental import pallas as pl
from jax.experimental.pallas import tpu as pltpu
from jax.experimental.pallas import tpu_sc as plsc

_NE = 8
_T = 2048
_D = 768
_F = 1536
_TM = 128
_PMAX = 5120
_NTMAX = _PMAX // _TM
_CH = 256
_NW = 32
_TPW = _T // _NW


def _routing_body(x_ref, rw_ref, pos0_ref, pos1_ref, cw0_ref, cw1_ref,
                  te_ref):
    x = x_ref[...]
    logits = jnp.dot(x, rw_ref[...], preferred_element_type=jnp.float32)
    iota_e = lax.broadcasted_iota(jnp.int32, (_T, _NE), 1)
    l1 = jnp.max(logits, axis=1, keepdims=True)
    i1 = jnp.min(jnp.where(logits == l1, iota_e, _NE), axis=1, keepdims=True)
    masked = jnp.where(iota_e == i1, -jnp.inf, logits)
    l2 = jnp.max(masked, axis=1, keepdims=True)
    i2 = jnp.min(jnp.where(masked == l2, iota_e, _NE), axis=1, keepdims=True)
    p1 = 1.0 / (1.0 + jnp.exp(l2 - l1))
    p2 = 1.0 - p1

    oh1 = (iota_e == i1).astype(jnp.float32)
    oh2 = (iota_e == i2).astype(jnp.float32)
    ohs = oh1 + oh2

    ii = lax.broadcasted_iota(jnp.int32, (_CH, _CH), 0)
    jj = lax.broadcasted_iota(jnp.int32, (_CH, _CH), 1)
    lts = (jj < ii).astype(jnp.float32)
    chunks = []
    carry = jnp.zeros((1, _NE), jnp.float32)
    for c in range(_T // _CH):
        blk = ohs[c * _CH:(c + 1) * _CH]
        chunks.append(jnp.dot(lts, blk, preferred_element_type=jnp.float32)
                      + carry)
        carry = carry + jnp.sum(blk, axis=0, keepdims=True)
    cum = jnp.concatenate(chunks, axis=0)
    counts = carry

    sizes = jnp.ceil(counts / _TM) * _TM
    ie = lax.broadcasted_iota(jnp.int32, (_NE, _NE), 0)
    je = lax.broadcasted_iota(jnp.int32, (_NE, _NE), 1)
    uts = (ie < je).astype(jnp.float32)
    start = jnp.dot(sizes, uts, preferred_element_type=jnp.float32)

    dest = start + cum
    pos0 = jnp.sum(oh1 * dest, axis=1)
    pos1 = jnp.sum(oh2 * dest, axis=1)
    pos0_ref[...] = pos0.astype(jnp.int32)
    pos1_ref[...] = pos1.astype(jnp.int32)
    cw0_ref[...] = p1[:, 0]
    cw1_ref[...] = p2[:, 0]

    tile_end = (start + sizes) * (1.0 / _TM)
    nf = lax.broadcasted_iota(jnp.int32, (_NTMAX, _NE), 0).astype(jnp.float32)
    te = jnp.sum((nf >= tile_end).astype(jnp.int32), axis=1)
    te_ref[...] = jnp.minimum(te, _NE - 1)


def _routing(x, router_w):
    return pl.pallas_call(
        _routing_body,
        grid=(1,),
        in_specs=[
            pl.BlockSpec((_T, _D), lambda i: (0, 0)),
            pl.BlockSpec((_D, _NE), lambda i: (0, 0)),
        ],
        out_specs=[
            pl.BlockSpec((_T,), lambda i: (0,)),
            pl.BlockSpec((_T,), lambda i: (0,)),
            pl.BlockSpec((_T,), lambda i: (0,)),
            pl.BlockSpec((_T,), lambda i: (0,)),
            pl.BlockSpec((_NTMAX,), lambda i: (0,)),
        ],
        out_shape=[
            jax.ShapeDtypeStruct((_T,), jnp.int32),
            jax.ShapeDtypeStruct((_T,), jnp.int32),
            jax.ShapeDtypeStruct((_T,), jnp.float32),
            jax.ShapeDtypeStruct((_T,), jnp.float32),
            jax.ShapeDtypeStruct((_NTMAX,), jnp.int32),
        ],
    )(x, router_w)


def _dispatch_body(x_hbm, pos0_hbm, pos1_hbm, xs_hbm,
                   idx0_v, idx1_v, rows_v, sem):
    wid = lax.axis_index("s") * 2 + lax.axis_index("c")
    base = wid * _TPW
    pltpu.sync_copy(pos0_hbm.at[pl.ds(base, _TPW)], idx0_v)
    pltpu.sync_copy(pos1_hbm.at[pl.ds(base, _TPW)], idx1_v)
    pltpu.sync_copy(x_hbm.at[pl.ds(base, _TPW)], rows_v)
    pltpu.async_copy(rows_v, xs_hbm.at[idx0_v], sem).wait()
    pltpu.async_copy(rows_v, xs_hbm.at[idx1_v], sem).wait()


def _dispatch(x_i32, pos0, pos1):
    mesh = plsc.VectorSubcoreMesh(core_axis_name="c", subcore_axis_name="s",
                                  num_cores=2, num_subcores=16)
    fn = pl.kernel(
        _dispatch_body,
        out_type=jax.ShapeDtypeStruct((_PMAX, _D // 2), jnp.int32),
        mesh=mesh,
        scratch_types=[
            pltpu.VMEM((_TPW,), jnp.int32),
            pltpu.VMEM((_TPW,), jnp.int32),
            pltpu.VMEM((_TPW, _D // 2), jnp.int32),
            pltpu.SemaphoreType.DMA,
        ],
    )
    return fn(x_i32, pos0, pos1)


def _ffn_body(te_ref, xs_ref, w0_ref, w1_ref, wo_ref, ys_ref):
    xb = xs_ref[...]
    hg = jnp.dot(xb, w0_ref[0], preferred_element_type=jnp.float32)
    hu = jnp.dot(xb, w1_ref[0], preferred_element_type=jnp.float32)
    act = (hg * jax.nn.sigmoid(hg) * hu).astype(jnp.bfloat16)
    ys_ref[...] = jnp.dot(act, wo_ref[0], preferred_element_type=jnp.float32)


def _grouped_ffn(tile_expert, xs, w0b, w1b, wob):
    grid_spec = pltpu.PrefetchScalarGridSpec(
        num_scalar_prefetch=1,
        grid=(_NTMAX,),
        in_specs=[
            pl.BlockSpec((_TM, _D), lambda n, te: (n, 0)),
            pl.BlockSpec((1, _D, _F), lambda n, te: (te[n], 0, 0)),
            pl.BlockSpec((1, _D, _F), lambda n, te: (te[n], 0, 0)),
            pl.BlockSpec((1, _F, _D), lambda n, te: (te[n], 0, 0)),
        ],
        out_specs=pl.BlockSpec((_TM, _D), lambda n, te: (n, 0)),
    )
    return pl.pallas_call(
        _ffn_body,
        grid_spec=grid_spec,
        out_shape=jax.ShapeDtypeStruct((_PMAX, _D), jnp.float32),
    )(tile_expert, xs, w0b, w1b, wob)


def _combine_body(ys_hbm, pos0_hbm, pos1_hbm, cw0_hbm, cw1_hbm, out_hbm,
                  idx0_v, idx1_v, b0, b1, w0v, w1v, sem):
    wid = lax.axis_index("s") * 2 + lax.axis_index("c")
    base = wid * _TPW
    pltpu.sync_copy(pos0_hbm.at[pl.ds(base, _TPW)], idx0_v)
    pltpu.sync_copy(pos1_hbm.at[pl.ds(base, _TPW)], idx1_v)
    pltpu.sync_copy(cw0_hbm.at[pl.ds(base, _TPW)], w0v)
    pltpu.sync_copy(cw1_hbm.at[pl.ds(base, _TPW)], w1v)
    pltpu.async_copy(ys_hbm.at[idx0_v], b0, sem).wait()
    pltpu.async_copy(ys_hbm.at[idx1_v], b1, sem).wait()

    def row_body(r, _):
        wa = plsc.load_gather(w0v, [jnp.full((16,), r, jnp.int32)])
        wb = plsc.load_gather(w1v, [jnp.full((16,), r, jnp.int32)])

        def seg_body(j, _):
            s0 = b0[r, pl.ds(j * 16, 16)]
            s1 = b1[r, pl.ds(j * 16, 16)]
            b0[r, pl.ds(j * 16, 16)] = s0 * wa + s1 * wb
            return 0

        lax.fori_loop(0, _D // 16, seg_body, 0)
        return 0

    lax.fori_loop(0, _TPW, row_body, 0)
    pltpu.sync_copy(b0, out_hbm.at[pl.ds(base, _TPW)])


def _combine(ys, pos0, pos1, cw0, cw1):
    mesh = plsc.VectorSubcoreMesh(core_axis_name="c", subcore_axis_name="s",
                                  num_cores=2, num_subcores=16)
    fn = pl.kernel(
        _combine_body,
        out_type=jax.ShapeDtypeStruct((_T, _D), jnp.float32),
        mesh=mesh,
        scratch_types=[
            pltpu.VMEM((_TPW,), jnp.int32),
            pltpu.VMEM((_TPW,), jnp.int32),
            pltpu.VMEM((_TPW, _D), jnp.float32),
            pltpu.VMEM((_TPW, _D), jnp.float32),
            pltpu.VMEM((_TPW,), jnp.float32),
            pltpu.VMEM((_TPW,), jnp.float32),
            pltpu.SemaphoreType.DMA,
        ],
        compiler_params=pltpu.CompilerParams(needs_layout_passes=False),
    )
    return fn(ys, pos0, pos1, cw0, cw1)


def kernel(hidden_states, router_w, w0, w1, wo):
    orig_shape = hidden_states.shape
    x = hidden_states.reshape(-1, orig_shape[-1])
    w0b = w0.astype(jnp.bfloat16)
    w1b = w1.astype(jnp.bfloat16)
    wob = wo.astype(jnp.bfloat16)

    pos0, pos1, cw0, cw1, tile_expert = _routing(x, router_w)
    x_i32 = lax.bitcast_convert_type(
        x.astype(jnp.bfloat16).reshape(_T, _D // 2, 2), jnp.int32)
    xs_i32 = _dispatch(x_i32, pos0, pos1)
    xs = lax.bitcast_convert_type(xs_i32, jnp.bfloat16).reshape(_PMAX, _D)
    ys = _grouped_ffn(tile_expert, xs, w0b, w1b, wob)
    out = _combine(ys, pos0, pos1, cw0, cw1)
    return out.reshape(orig_shape)


# speedup vs baseline: 1.9679x; 1.9679x over previous
import functools

import jax
import jax.numpy as jnp
from jax import lax
from jax.experimental import pallas as pl
from jax.experimental.pallas import tpu as pltpu
from jax.experimental.pallas import tpu_sc as plsc

_NE = 8
_T = 2048
_D = 768
_F = 1536
_TM = 128
_PMAX = 5120
_NTMAX = _PMAX // _TM
_CH = 256
_NW = 32
_TPW = _T // _NW


def _routing_body(x_ref, rw_ref, pos0_ref, pos1_ref, cw0_ref, cw1_ref,
                  te_ref):
    x = x_ref[...]
    logits = jnp.dot(x, rw_ref[...], preferred_element_type=jnp.float32)
    iota_e = lax.broadcasted_iota(jnp.int32, (_T, _NE), 1)
    l1 = jnp.max(logits, axis=1, keepdims=True)
    i1 = jnp.min(jnp.where(logits == l1, iota_e, _NE), axis=1, keepdims=True)
    masked = jnp.where(iota_e == i1, -jnp.inf, logits)
    l2 = jnp.max(masked, axis=1, keepdims=True)
    i2 = jnp.min(jnp.where(masked == l2, iota_e, _NE), axis=1, keepdims=True)
    p1 = 1.0 / (1.0 + jnp.exp(l2 - l1))
    p2 = 1.0 - p1

    oh1 = (iota_e == i1).astype(jnp.float32)
    oh2 = (iota_e == i2).astype(jnp.float32)
    ohs = oh1 + oh2

    ii = lax.broadcasted_iota(jnp.int32, (_CH, _CH), 0)
    jj = lax.broadcasted_iota(jnp.int32, (_CH, _CH), 1)
    lts = (jj < ii).astype(jnp.float32)
    chunks = []
    carry = jnp.zeros((1, _NE), jnp.float32)
    for c in range(_T // _CH):
        blk = ohs[c * _CH:(c + 1) * _CH]
        chunks.append(jnp.dot(lts, blk, preferred_element_type=jnp.float32)
                      + carry)
        carry = carry + jnp.sum(blk, axis=0, keepdims=True)
    cum = jnp.concatenate(chunks, axis=0)
    counts = carry

    sizes = jnp.ceil(counts / _TM) * _TM
    ie = lax.broadcasted_iota(jnp.int32, (_NE, _NE), 0)
    je = lax.broadcasted_iota(jnp.int32, (_NE, _NE), 1)
    uts = (ie < je).astype(jnp.float32)
    start = jnp.dot(sizes, uts, preferred_element_type=jnp.float32)

    dest = start + cum
    pos0 = jnp.sum(oh1 * dest, axis=1)
    pos1 = jnp.sum(oh2 * dest, axis=1)
    pos0_ref[...] = pos0.astype(jnp.int32)
    pos1_ref[...] = pos1.astype(jnp.int32)
    cw0_ref[...] = p1[:, 0]
    cw1_ref[...] = p2[:, 0]

    tile_end = (start + sizes) * (1.0 / _TM)
    nf = lax.broadcasted_iota(jnp.int32, (_NTMAX, _NE), 0).astype(jnp.float32)
    te = jnp.sum((nf >= tile_end).astype(jnp.int32), axis=1)
    te_ref[...] = jnp.minimum(te, _NE - 1)


def _routing(x, router_w):
    return pl.pallas_call(
        _routing_body,
        grid=(1,),
        in_specs=[
            pl.BlockSpec((_T, _D), lambda i: (0, 0)),
            pl.BlockSpec((_D, _NE), lambda i: (0, 0)),
        ],
        out_specs=[
            pl.BlockSpec((_T,), lambda i: (0,)),
            pl.BlockSpec((_T,), lambda i: (0,)),
            pl.BlockSpec((_T,), lambda i: (0,)),
            pl.BlockSpec((_T,), lambda i: (0,)),
            pl.BlockSpec((_NTMAX,), lambda i: (0,)),
        ],
        out_shape=[
            jax.ShapeDtypeStruct((_T,), jnp.int32),
            jax.ShapeDtypeStruct((_T,), jnp.int32),
            jax.ShapeDtypeStruct((_T,), jnp.float32),
            jax.ShapeDtypeStruct((_T,), jnp.float32),
            jax.ShapeDtypeStruct((_NTMAX,), jnp.int32),
        ],
    )(x, router_w)


def _dispatch_body(x_hbm, pos0_hbm, pos1_hbm, xs_hbm,
                   idx0_v, idx1_v, rows_v, sem):
    wid = lax.axis_index("s") * 2 + lax.axis_index("c")
    base = wid * _TPW
    pltpu.sync_copy(pos0_hbm.at[pl.ds(base, _TPW)], idx0_v)
    pltpu.sync_copy(pos1_hbm.at[pl.ds(base, _TPW)], idx1_v)
    pltpu.sync_copy(x_hbm.at[pl.ds(base, _TPW)], rows_v)
    pltpu.async_copy(rows_v, xs_hbm.at[idx0_v], sem).wait()
    pltpu.async_copy(rows_v, xs_hbm.at[idx1_v], sem).wait()


def _dispatch(x, pos0, pos1):
    mesh = plsc.VectorSubcoreMesh(core_axis_name="c", subcore_axis_name="s",
                                  num_cores=2, num_subcores=16)
    fn = pl.kernel(
        _dispatch_body,
        out_type=jax.ShapeDtypeStruct((_PMAX, _D), jnp.float32),
        mesh=mesh,
        scratch_types=[
            pltpu.VMEM((_TPW,), jnp.int32),
            pltpu.VMEM((_TPW,), jnp.int32),
            pltpu.VMEM((_TPW, _D), jnp.float32),
            pltpu.SemaphoreType.DMA,
        ],
    )
    return fn(x, pos0, pos1)


def _ffn_body(te_ref, xs_ref, w0_ref, w1_ref, wo_ref, ys_ref):
    xb = xs_ref[...].astype(jnp.bfloat16)
    hg = jnp.dot(xb, w0_ref[0].astype(jnp.bfloat16),
                 preferred_element_type=jnp.float32)
    hu = jnp.dot(xb, w1_ref[0].astype(jnp.bfloat16),
                 preferred_element_type=jnp.float32)
    act = (hg * jax.nn.sigmoid(hg) * hu).astype(jnp.bfloat16)
    ys_ref[...] = jnp.dot(act, wo_ref[0].astype(jnp.bfloat16),
                          preferred_element_type=jnp.float32)


def _grouped_ffn(tile_expert, xs, w0b, w1b, wob):
    grid_spec = pltpu.PrefetchScalarGridSpec(
        num_scalar_prefetch=1,
        grid=(_NTMAX,),
        in_specs=[
            pl.BlockSpec((_TM, _D), lambda n, te: (n, 0)),
            pl.BlockSpec((1, _D, _F), lambda n, te: (te[n], 0, 0)),
            pl.BlockSpec((1, _D, _F), lambda n, te: (te[n], 0, 0)),
            pl.BlockSpec((1, _F, _D), lambda n, te: (te[n], 0, 0)),
        ],
        out_specs=pl.BlockSpec((_TM, _D), lambda n, te: (n, 0)),
    )
    return pl.pallas_call(
        _ffn_body,
        grid_spec=grid_spec,
        out_shape=jax.ShapeDtypeStruct((_PMAX, _D), jnp.float32),
    )(tile_expert, xs, w0b, w1b, wob)


def _combine_body(ys_hbm, pos0_hbm, pos1_hbm, cw0_hbm, cw1_hbm, out_hbm,
                  idx0_v, idx1_v, b0, b1, w0v, w1v, sem):
    wid = lax.axis_index("s") * 2 + lax.axis_index("c")
    base = wid * _TPW
    pltpu.sync_copy(pos0_hbm.at[pl.ds(base, _TPW)], idx0_v)
    pltpu.sync_copy(pos1_hbm.at[pl.ds(base, _TPW)], idx1_v)
    pltpu.sync_copy(cw0_hbm.at[pl.ds(base, _TPW)], w0v)
    pltpu.sync_copy(cw1_hbm.at[pl.ds(base, _TPW)], w1v)
    pltpu.async_copy(ys_hbm.at[idx0_v], b0, sem).wait()
    pltpu.async_copy(ys_hbm.at[idx1_v], b1, sem).wait()

    def row_body(r, _):
        wa = plsc.load_gather(w0v, [jnp.full((16,), r, jnp.int32)])
        wb = plsc.load_gather(w1v, [jnp.full((16,), r, jnp.int32)])

        def seg_body(j, _):
            s0 = b0[r, pl.ds(j * 16, 16)]
            s1 = b1[r, pl.ds(j * 16, 16)]
            b0[r, pl.ds(j * 16, 16)] = s0 * wa + s1 * wb
            return 0

        lax.fori_loop(0, _D // 16, seg_body, 0)
        return 0

    lax.fori_loop(0, _TPW, row_body, 0)
    pltpu.sync_copy(b0, out_hbm.at[pl.ds(base, _TPW)])


def _combine(ys, pos0, pos1, cw0, cw1):
    mesh = plsc.VectorSubcoreMesh(core_axis_name="c", subcore_axis_name="s",
                                  num_cores=2, num_subcores=16)
    fn = pl.kernel(
        _combine_body,
        out_type=jax.ShapeDtypeStruct((_T, _D), jnp.float32),
        mesh=mesh,
        scratch_types=[
            pltpu.VMEM((_TPW,), jnp.int32),
            pltpu.VMEM((_TPW,), jnp.int32),
            pltpu.VMEM((_TPW, _D), jnp.float32),
            pltpu.VMEM((_TPW, _D), jnp.float32),
            pltpu.VMEM((_TPW,), jnp.float32),
            pltpu.VMEM((_TPW,), jnp.float32),
            pltpu.SemaphoreType.DMA,
        ],
        compiler_params=pltpu.CompilerParams(needs_layout_passes=False),
    )
    return fn(ys, pos0, pos1, cw0, cw1)


def kernel(hidden_states, router_w, w0, w1, wo):
    orig_shape = hidden_states.shape
    x = hidden_states.reshape(-1, orig_shape[-1])

    pos0, pos1, cw0, cw1, tile_expert = _routing(x, router_w)
    xs = _dispatch(x, pos0, pos1)
    ys = _grouped_ffn(tile_expert, xs, w0, w1, wo)
    out = _combine(ys, pos0, pos1, cw0, cw1)
    return out.reshape(orig_shape)
